# baseline (speedup 1.0000x reference)
import jax
import jax.numpy as jnp
from jax.experimental import pallas as pl


_BATCH_BLOCK = 128


def _broadcast_body(pos_emb_ref, out_ref):
    out_ref[...] = jnp.broadcast_to(pos_emb_ref[...][None], out_ref.shape)


def kernel(x, pos_emb):
    batch = x.shape[0]
    seq, dim = pos_emb.shape
    grid = batch // _BATCH_BLOCK
    return pl.pallas_call(
        _broadcast_body,
        grid=(grid,),
        in_specs=[pl.BlockSpec((seq, dim), lambda i: (0, 0))],
        out_specs=pl.BlockSpec((_BATCH_BLOCK, seq, dim), lambda i: (i, 0, 0)),
        out_shape=jax.ShapeDtypeStruct((batch, seq, dim), jnp.float32),
    )(pos_emb)


# speedup vs baseline: 1.0631x; 1.0631x over previous
import jax
import jax.numpy as jnp
from jax.experimental import pallas as pl


_BATCH_BLOCK = 32


def _broadcast_body(pos_emb_ref, out_ref):
    out_ref[...] = jnp.broadcast_to(pos_emb_ref[...][None], out_ref.shape)


def kernel(x, pos_emb):
    batch = x.shape[0]
    seq, dim = pos_emb.shape
    grid = batch // _BATCH_BLOCK
    return pl.pallas_call(
        _broadcast_body,
        grid=(grid,),
        in_specs=[pl.BlockSpec((seq, dim), lambda i: (0, 0))],
        out_specs=pl.BlockSpec((_BATCH_BLOCK, seq, dim), lambda i: (i, 0, 0)),
        out_shape=jax.ShapeDtypeStruct((batch, seq, dim), jnp.float32),
    )(pos_emb)
